# Initial kernel scaffold; baseline (speedup 1.0000x reference)
import jax
import jax.numpy as jnp
from jax.experimental import pallas as pl


def kernel(edge_index, edge_attr, node_emb, r_emb, r_proj):
    raise NotImplementedError("write your pallas kernel here")


# speedup vs baseline: 1.5947x; 1.5947x over previous
import functools

import jax
import jax.numpy as jnp
from jax import lax
from jax.experimental import pallas as pl
from jax.experimental.pallas import tpu as pltpu
from jax.experimental.pallas import tpu_sc as plsc

EMB = 32
_NW = 32
_CHUNK = 128
_BLK = 1024


def _sc_gather(idx3d, table, total_rows, nch):
    mesh = plsc.VectorSubcoreMesh(core_axis_name="c", subcore_axis_name="s")

    @functools.partial(
        pl.kernel,
        out_type=jax.ShapeDtypeStruct((total_rows, EMB), jnp.float32),
        mesh=mesh,
        scratch_types=[
            pltpu.VMEM((nch, _CHUNK), jnp.int32),
            pltpu.VMEM((_CHUNK, EMB), jnp.float32),
            pltpu.SemaphoreType.DMA,
        ],
        compiler_params=pltpu.CompilerParams(use_tc_tiling_on_sc=False),
    )
    def gather_kernel(idx_hbm, table_hbm, out_hbm, idx_v, rows_v, sem):
        wid = lax.axis_index("s") * 2 + lax.axis_index("c")
        pltpu.sync_copy(idx_hbm.at[wid], idx_v)
        base = wid * (nch * _CHUNK)

        def body(j, carry):
            pltpu.async_copy(table_hbm.at[idx_v.at[j]], rows_v, sem).wait()
            row = pl.multiple_of(base + j * _CHUNK, _CHUNK)
            pltpu.sync_copy(rows_v, out_hbm.at[pl.ds(row, _CHUNK)])
            return carry

        lax.fori_loop(0, nch, body, 0)

    return gather_kernel(idx3d, table)


def _loss_body(e_true, r_ref, h_ref, t_ref, pf_ref, re_ref, o_ref):
    i = pl.program_id(0)
    nb = pl.num_programs(0)
    d = h_ref[...] - t_ref[...]
    r = r_ref[...]
    rowid = i * _BLK + lax.broadcasted_iota(jnp.int32, (_BLK, 1), 0)
    valid = rowid < e_true
    g = r >> 1

    cols = lax.broadcasted_iota(jnp.int32, (_BLK, 32 * EMB), 1)
    d_tiled = jnp.concatenate([d] * 32, axis=1)
    z = jnp.where(((cols >> 5) == g) & valid, d_tiled, 0.0)
    v = lax.dot_general(z, pf_ref[...], (((1,), (0,)), ((), ())),
                        preferred_element_type=jnp.float32)

    rcols = lax.broadcasted_iota(jnp.int32, (_BLK, 64), 1)
    oh = jnp.where((rcols == r) & valid, 1.0, 0.0)
    r_e = lax.dot_general(oh, re_ref[...], (((1,), (0,)), ((), ())),
                          preferred_element_type=jnp.float32)

    u = v + r_e
    s = jnp.reshape(jnp.sum(u * u), (1, 1))

    @pl.when(i == 0)
    def _():
        o_ref[...] = jnp.zeros((1, 1), jnp.float32)

    o_ref[...] += s

    @pl.when(i == nb - 1)
    def _():
        o_ref[...] = o_ref[...] * (1.0 / (e_true * EMB))


def kernel(edge_index, edge_attr, node_emb, r_emb, r_proj):
    e_true = edge_index.shape[1]
    e_pad = ((e_true + 2047) // 2048) * 2048
    nch = (2 * e_pad) // (_NW * _CHUNK)
    nb = e_pad // _BLK

    heads = jnp.pad(edge_index[0].astype(jnp.int32), (0, e_pad - e_true))
    tails = jnp.pad(edge_index[1].astype(jnp.int32), (0, e_pad - e_true))
    all_idx = jnp.concatenate([heads, tails]).reshape(_NW, nch, _CHUNK)

    gath = _sc_gather(all_idx, node_emb, 2 * e_pad, nch)

    r_pad = jnp.pad(edge_attr.astype(jnp.int32), ((0, e_pad - e_true), (0, 0)))
    pf = r_proj.reshape(EMB * EMB, EMB)

    loss = pl.pallas_call(
        functools.partial(_loss_body, e_true),
        grid=(nb,),
        in_specs=[
            pl.BlockSpec((_BLK, 1), lambda i: (i, 0)),
            pl.BlockSpec((_BLK, EMB), lambda i: (i, 0)),
            pl.BlockSpec((_BLK, EMB), lambda i: (i + nb, 0)),
            pl.BlockSpec((EMB * EMB, EMB), lambda i: (0, 0)),
            pl.BlockSpec((64, EMB), lambda i: (0, 0)),
        ],
        out_specs=pl.BlockSpec((1, 1), lambda i: (0, 0)),
        out_shape=jax.ShapeDtypeStruct((1, 1), jnp.float32),
    )(r_pad, gath, gath, pf, r_emb)

    return jnp.reshape(loss, ())


# speedup vs baseline: 1.7686x; 1.1091x over previous
import functools

import jax
import jax.numpy as jnp
from jax import lax
from jax.experimental import pallas as pl
from jax.experimental.pallas import tpu as pltpu
from jax.experimental.pallas import tpu_sc as plsc

EMB = 32
_NW = 32
_CHUNK = 128
_BLK = 2048
_SUB = _BLK // 4


def _sc_gather(idx3d, table, total_rows, nch):
    mesh = plsc.VectorSubcoreMesh(core_axis_name="c", subcore_axis_name="s")

    @functools.partial(
        pl.kernel,
        out_type=jax.ShapeDtypeStruct((total_rows, EMB), jnp.float32),
        mesh=mesh,
        scratch_types=[
            pltpu.VMEM((nch, _CHUNK), jnp.int32),
            pltpu.VMEM((2, _CHUNK, EMB), jnp.float32),
            pltpu.SemaphoreType.DMA,
        ],
        compiler_params=pltpu.CompilerParams(use_tc_tiling_on_sc=False),
    )
    def gather_kernel(idx_hbm, table_hbm, out_hbm, idx_v, rows_v, gsem):
        wid = lax.axis_index("s") * 2 + lax.axis_index("c")
        pltpu.sync_copy(idx_hbm.at[wid], idx_v)
        base = wid * (nch * _CHUNK)

        pltpu.async_copy(table_hbm.at[idx_v.at[0]], rows_v.at[0], gsem)

        def body(j, carry):
            buf = rows_v.at[j % 2]
            pltpu.make_async_copy(table_hbm.at[idx_v.at[j]], buf, gsem).wait()

            @pl.when(j + 1 < nch)
            def _():
                pltpu.async_copy(
                    table_hbm.at[idx_v.at[j + 1]], rows_v.at[(j + 1) % 2], gsem
                )

            row = pl.multiple_of(base + j * _CHUNK, _CHUNK)
            pltpu.sync_copy(buf, out_hbm.at[pl.ds(row, _CHUNK)])
            return carry

        lax.fori_loop(0, nch, body, 0)

    return gather_kernel(idx3d, table)


def _loss_body(e_true, r_ref, h_ref, t_ref, pf_ref, re_ref, o_ref):
    i = pl.program_id(0)
    nb = pl.num_programs(0)
    dp = h_ref[...] - t_ref[...]
    s = jnp.zeros((), jnp.float32)
    for k in range(4):
        d = dp[:, 32 * k:32 * (k + 1)]
        r = r_ref[pl.ds(k * _SUB, _SUB), :]
        rowid = i * _BLK + 4 * lax.broadcasted_iota(jnp.int32, (_SUB, 1), 0) + k
        valid = rowid < e_true
        g = r >> 1

        cols = lax.broadcasted_iota(jnp.int32, (_SUB, 32 * EMB), 1)
        d_tiled = jnp.concatenate([d] * 32, axis=1)
        z = jnp.where(((cols >> 5) == g) & valid, d_tiled, 0.0)
        v = lax.dot_general(z, pf_ref[...], (((1,), (0,)), ((), ())),
                            preferred_element_type=jnp.float32)

        rcols = lax.broadcasted_iota(jnp.int32, (_SUB, 64), 1)
        oh = jnp.where((rcols == r) & valid, 1.0, 0.0)
        r_e = lax.dot_general(oh, re_ref[...], (((1,), (0,)), ((), ())),
                              preferred_element_type=jnp.float32)

        u = v + r_e
        s = s + jnp.sum(u * u)

    @pl.when(i == 0)
    def _():
        o_ref[...] = jnp.zeros((1, 1), jnp.float32)

    o_ref[...] += jnp.reshape(s, (1, 1))

    @pl.when(i == nb - 1)
    def _():
        o_ref[...] = o_ref[...] * (1.0 / (e_true * EMB))


def kernel(edge_index, edge_attr, node_emb, r_emb, r_proj):
    e_true = edge_index.shape[1]
    e_pad = ((e_true + 2047) // 2048) * 2048
    nch = (2 * e_pad) // (_NW * _CHUNK)
    nb = e_pad // _BLK

    heads = jnp.pad(edge_index[0].astype(jnp.int32), (0, e_pad - e_true))
    tails = jnp.pad(edge_index[1].astype(jnp.int32), (0, e_pad - e_true))
    all_idx = jnp.concatenate([heads, tails]).reshape(_NW, nch, _CHUNK)

    gath = _sc_gather(all_idx, node_emb, 2 * e_pad, nch)
    gath4 = gath.reshape(e_pad // 2, 4 * EMB)

    r_pad = jnp.pad(edge_attr[:, 0].astype(jnp.int32), (0, e_pad - e_true))
    r4 = r_pad.reshape(nb, _SUB, 4).transpose(0, 2, 1).reshape(e_pad, 1)

    pf = r_proj.reshape(EMB * EMB, EMB)

    nrow = _BLK // 4
    loss = pl.pallas_call(
        functools.partial(_loss_body, e_true),
        grid=(nb,),
        in_specs=[
            pl.BlockSpec((_BLK, 1), lambda i: (i, 0)),
            pl.BlockSpec((nrow, 4 * EMB), lambda i: (i, 0)),
            pl.BlockSpec((nrow, 4 * EMB), lambda i: (i + nb, 0)),
            pl.BlockSpec((EMB * EMB, EMB), lambda i: (0, 0)),
            pl.BlockSpec((64, EMB), lambda i: (0, 0)),
        ],
        out_specs=pl.BlockSpec((1, 1), lambda i: (0, 0)),
        out_shape=jax.ShapeDtypeStruct((1, 1), jnp.float32),
    )(r4, gath4, gath4, pf, r_emb)

    return jnp.reshape(loss, ())


# speedup vs baseline: 1.8921x; 1.0699x over previous
import functools

import jax
import jax.numpy as jnp
from jax import lax
from jax.experimental import pallas as pl
from jax.experimental.pallas import tpu as pltpu
from jax.experimental.pallas import tpu_sc as plsc

EMB = 32
_NW = 32
_CHUNK = 128
_BLK = 2048
_SUB = _BLK // 4


def _sc_gather(idx3d, table_flat, total_rows, nch):
    mesh = plsc.VectorSubcoreMesh(core_axis_name="c", subcore_axis_name="s")

    @functools.partial(
        pl.kernel,
        out_type=jax.ShapeDtypeStruct((total_rows, EMB), jnp.float32),
        mesh=mesh,
        scratch_types=[
            pltpu.VMEM((nch, _CHUNK), jnp.int32),
            pltpu.VMEM((2, _CHUNK, EMB), jnp.float32),
            pltpu.SemaphoreType.DMA,
        ],
        compiler_params=pltpu.CompilerParams(use_tc_tiling_on_sc=False),
    )
    def gather_kernel(idx_hbm, table_hbm, out_hbm, idx_v, rows_v, gsem):
        wid = lax.axis_index("s") * 2 + lax.axis_index("c")
        pltpu.sync_copy(idx_hbm.at[wid], idx_v)
        base = wid * (nch * _CHUNK)

        pltpu.async_copy(table_hbm.at[idx_v.at[0]], rows_v.at[0], gsem)

        def body(j, carry):
            buf = rows_v.at[j % 2]
            pltpu.make_async_copy(table_hbm.at[idx_v.at[j]], buf, gsem).wait()

            @pl.when(j + 1 < nch)
            def _():
                pltpu.async_copy(
                    table_hbm.at[idx_v.at[j + 1]], rows_v.at[(j + 1) % 2], gsem
                )

            row = pl.multiple_of(base + j * _CHUNK, _CHUNK)
            pltpu.sync_copy(buf, out_hbm.at[pl.ds(row, _CHUNK)])
            return carry

        lax.fori_loop(0, nch, body, 0)

    return gather_kernel(idx3d, table_flat)


def _loss_body(e_true, r_ref, h_ref, t_ref, pf_ref, re_ref, o_ref):
    i = pl.program_id(0)
    nb = pl.num_programs(0)
    dp = h_ref[...] - t_ref[...]
    s = jnp.zeros((), jnp.float32)
    for k in range(4):
        d = dp[:, 32 * k:32 * (k + 1)]
        r = r_ref[:, k:k + 1]
        rowid = i * _BLK + 4 * lax.broadcasted_iota(jnp.int32, (_SUB, 1), 0) + k
        valid = rowid < e_true
        g = r >> 1

        cols = lax.broadcasted_iota(jnp.int32, (_SUB, 32 * EMB), 1)
        d_tiled = jnp.concatenate([d] * 32, axis=1)
        z = jnp.where(((cols >> 5) == g) & valid, d_tiled, 0.0)
        v = lax.dot_general(z, pf_ref[...], (((1,), (0,)), ((), ())),
                            preferred_element_type=jnp.float32)

        rcols = lax.broadcasted_iota(jnp.int32, (_SUB, 64), 1)
        oh = jnp.where((rcols == r) & valid, 1.0, 0.0)
        r_e = lax.dot_general(oh, re_ref[...], (((1,), (0,)), ((), ())),
                              preferred_element_type=jnp.float32)

        u = v + r_e
        s = s + jnp.sum(u * u)

    @pl.when(i == 0)
    def _():
        o_ref[...] = jnp.zeros((1, 1), jnp.float32)

    o_ref[...] += jnp.reshape(s, (1, 1))

    @pl.when(i == nb - 1)
    def _():
        o_ref[...] = o_ref[...] * (1.0 / (e_true * EMB))


def kernel(edge_index, edge_attr, node_emb, r_emb, r_proj):
    e_true = edge_index.shape[1]
    e_pad = ((e_true + 2047) // 2048) * 2048
    nch = (2 * e_pad) // (_NW * _CHUNK)
    nb = e_pad // _BLK

    heads = jnp.pad(edge_index[0].astype(jnp.int32), (0, e_pad - e_true))
    tails = jnp.pad(edge_index[1].astype(jnp.int32), (0, e_pad - e_true))
    all_idx = jnp.concatenate([heads, tails]).reshape(_NW, nch, _CHUNK)

    nodes = node_emb.shape[0]
    nm1d = lax.optimization_barrier(jnp.reshape(node_emb, (-1,)))
    gath = _sc_gather(all_idx, nm1d.reshape(nodes, EMB), 2 * e_pad, nch)
    gath4 = gath.reshape(e_pad // 2, 4 * EMB)

    r_pad = jnp.pad(edge_attr[:, 0].astype(jnp.int32), (0, e_pad - e_true))
    r2 = r_pad.reshape(e_pad // 4, 4)

    pf = r_proj.reshape(EMB * EMB, EMB)

    nrow = _BLK // 4
    loss = pl.pallas_call(
        functools.partial(_loss_body, e_true),
        grid=(nb,),
        in_specs=[
            pl.BlockSpec((nrow, 4), lambda i: (i, 0)),
            pl.BlockSpec((nrow, 4 * EMB), lambda i: (i, 0)),
            pl.BlockSpec((nrow, 4 * EMB), lambda i: (i + nb, 0)),
            pl.BlockSpec((EMB * EMB, EMB), lambda i: (0, 0)),
            pl.BlockSpec((64, EMB), lambda i: (0, 0)),
        ],
        out_specs=pl.BlockSpec((1, 1), lambda i: (0, 0)),
        out_shape=jax.ShapeDtypeStruct((1, 1), jnp.float32),
    )(r2, gath4, gath4, pf, r_emb)

    return jnp.reshape(loss, ())
